# baseline (speedup 1.0000x reference)
import functools

import jax
import jax.numpy as jnp
from jax import lax
from jax.experimental import pallas as pl
from jax.experimental.pallas import tpu as pltpu
from jax.experimental.pallas import tpu_sc as plsc

M_TILE = 512


def _prep_body(emb_ref, embbf_ref, e2_ref):
    emb = emb_ref[...]
    embbf_ref[...] = emb.astype(jnp.bfloat16)
    e2_ref[...] = jnp.sum(emb * emb, axis=1)


def _prep_call(embedding):
    k, d = embedding.shape
    return pl.pallas_call(
        _prep_body,
        out_shape=[
            jax.ShapeDtypeStruct((k, d), jnp.bfloat16),
            jax.ShapeDtypeStruct((k,), jnp.float32),
        ],
    )(embedding)


def _argmin_body(z_ref, embbf_ref, e2_ref, ind_ref, loss_ref, acc_ref):
    i = pl.program_id(0)
    z = z_ref[...]
    embbf = embbf_ref[...]
    k = embbf.shape[0]

    @pl.when(i == 0)
    def _():
        acc_ref[0, 0] = 0.0

    zm2 = (z * -2.0).astype(jnp.bfloat16)
    scores_m2 = lax.dot_general(
        zm2, embbf,
        dimension_numbers=(((1,), (1,)), ((), ())),
        preferred_element_type=jnp.float32,
    )
    rowz2 = jnp.sum(z * z, axis=1)
    e2r = e2_ref[...][None, :]
    dist = rowz2[:, None] + scores_m2 + e2r
    minval = jnp.min(dist, axis=1, keepdims=True)
    idsf = lax.broadcasted_iota(jnp.int32, dist.shape, 1).astype(jnp.float32)
    indf = jnp.min(jnp.where(dist == minval, idsf, jnp.float32(k)), axis=1)
    ind_ref[...] = indf.astype(jnp.int32)
    acc_ref[0, 0] += jnp.sum(minval[:, 0])

    @pl.when(i == pl.num_programs(0) - 1)
    def _():
        loss_ref[0, 0] = acc_ref[0, 0]


def _argmin_call(z_e, embbf, e2):
    bn, d = z_e.shape
    k = embbf.shape[0]
    grid = bn // M_TILE
    return pl.pallas_call(
        _argmin_body,
        grid=(grid,),
        in_specs=[
            pl.BlockSpec((M_TILE, d), lambda i: (i, 0)),
            pl.BlockSpec((k, d), lambda i: (0, 0)),
            pl.BlockSpec((k,), lambda i: (0,)),
        ],
        out_specs=[
            pl.BlockSpec((M_TILE,), lambda i: (i,)),
            pl.BlockSpec(memory_space=pltpu.SMEM),
        ],
        out_shape=[
            jax.ShapeDtypeStruct((bn,), jnp.int32),
            jax.ShapeDtypeStruct((1, 1), jnp.float32),
        ],
        scratch_shapes=[
            pltpu.SMEM((1, 1), jnp.float32),
        ],
    )(z_e, embbf, e2)


_SC_CHUNK = 128


def _sc_gather(embedding, ind):
    bn = ind.shape[0]
    k, d = embedding.shape
    info = plsc.get_sparse_core_info()
    nw = info.num_cores * info.num_subcores
    b_per_w = bn // nw
    n_chunks = b_per_w // _SC_CHUNK
    mesh = plsc.VectorSubcoreMesh(core_axis_name="c", subcore_axis_name="s")

    @functools.partial(
        pl.kernel,
        mesh=mesh,
        out_type=jax.ShapeDtypeStruct((bn, d), jnp.float32),
        scratch_types=[
            pltpu.VMEM((_SC_CHUNK,), jnp.int32),
            pltpu.VMEM((_SC_CHUNK,), jnp.int32),
            pltpu.VMEM((_SC_CHUNK, d), jnp.float32),
            pltpu.VMEM((_SC_CHUNK, d), jnp.float32),
            pltpu.SemaphoreType.DMA,
            pltpu.SemaphoreType.DMA,
        ],
    )
    def gather_kernel(table_hbm, idx_hbm, out_hbm,
                      idx_a, idx_b, rows_a, rows_b, sem_a, sem_b):
        wid = lax.axis_index("s") * info.num_cores + lax.axis_index("c")
        base = wid * b_per_w
        idx_v = (idx_a, idx_b)
        rows_v = (rows_a, rows_b)
        sems = (sem_a, sem_b)
        pltpu.sync_copy(idx_hbm.at[pl.ds(base, _SC_CHUNK)], idx_a)
        pltpu.async_copy(table_hbm.at[idx_a], rows_a, sem_a)
        for c in range(n_chunks):
            cur = c % 2
            nxt = (c + 1) % 2
            if c + 1 < n_chunks:
                off_n = base + (c + 1) * _SC_CHUNK
                pltpu.sync_copy(idx_hbm.at[pl.ds(off_n, _SC_CHUNK)],
                                idx_v[nxt])
                pltpu.async_copy(table_hbm.at[idx_v[nxt]], rows_v[nxt],
                                 sems[nxt])
            pltpu.make_async_copy(table_hbm.at[idx_v[cur]], rows_v[cur],
                                  sems[cur]).wait()
            off = base + c * _SC_CHUNK
            pltpu.sync_copy(rows_v[cur], out_hbm.at[pl.ds(off, _SC_CHUNK)])

    return gather_kernel(embedding, ind)


def kernel(z, embedding):
    b, n, d = z.shape
    bn = b * n
    z_e = z.reshape(bn, d)
    embbf, e2 = _prep_call(embedding)
    h = bn // 2
    ind0, ls0 = _argmin_call(z_e[:h], embbf, e2)
    z_q0 = _sc_gather(embedding, ind0)
    ind1, ls1 = _argmin_call(z_e[h:], embbf, e2)
    z_q1 = _sc_gather(embedding, ind1)
    latent_loss = (ls0[0, 0] + ls1[0, 0]) * (12.5 / (bn * d))
    out = jnp.concatenate([z_q0, z_q1], axis=0).reshape(b, n * d)
    return (out, latent_loss)


# speedup vs baseline: 1.1656x; 1.1656x over previous
import functools

import jax
import jax.numpy as jnp
from jax import lax
from jax.experimental import pallas as pl
from jax.experimental.pallas import tpu as pltpu
from jax.experimental.pallas import tpu_sc as plsc

M_TILE = 1024


def _prep_body(emb_ref, embbf_ref, e2_ref):
    emb = emb_ref[...]
    embbf_ref[...] = emb.astype(jnp.bfloat16)
    e2_ref[...] = jnp.sum(emb * emb, axis=1)


def _prep_call(embedding):
    k, d = embedding.shape
    return pl.pallas_call(
        _prep_body,
        out_shape=[
            jax.ShapeDtypeStruct((k, d), jnp.bfloat16),
            jax.ShapeDtypeStruct((k,), jnp.float32),
        ],
    )(embedding)


def _argmin_body(z_ref, embbf_ref, e2_ref, ind_ref, loss_ref, acc_ref):
    i = pl.program_id(0)
    z = z_ref[...]
    embbf = embbf_ref[...]
    k = embbf.shape[0]

    @pl.when(i == 0)
    def _():
        acc_ref[0, 0] = 0.0

    zm2 = (z * -2.0).astype(jnp.bfloat16)
    scores_m2 = lax.dot_general(
        zm2, embbf,
        dimension_numbers=(((1,), (1,)), ((), ())),
        preferred_element_type=jnp.float32,
    )
    rowz2 = jnp.sum(z * z, axis=1)
    e2r = e2_ref[...][None, :]
    dist = rowz2[:, None] + scores_m2 + e2r
    minval = jnp.min(dist, axis=1, keepdims=True)
    idsf = lax.broadcasted_iota(jnp.int32, dist.shape, 1).astype(jnp.float32)
    indf = jnp.min(jnp.where(dist == minval, idsf, jnp.float32(k)), axis=1)
    ind_ref[...] = indf.astype(jnp.int32)
    acc_ref[0, 0] += jnp.sum(minval[:, 0])

    @pl.when(i == pl.num_programs(0) - 1)
    def _():
        loss_ref[0, 0] = acc_ref[0, 0]


def _argmin_call(z_e, embbf, e2):
    bn, d = z_e.shape
    k = embbf.shape[0]
    grid = bn // M_TILE
    return pl.pallas_call(
        _argmin_body,
        grid=(grid,),
        in_specs=[
            pl.BlockSpec((M_TILE, d), lambda i: (i, 0)),
            pl.BlockSpec((k, d), lambda i: (0, 0)),
            pl.BlockSpec((k,), lambda i: (0,)),
        ],
        out_specs=[
            pl.BlockSpec((M_TILE,), lambda i: (i,)),
            pl.BlockSpec(memory_space=pltpu.SMEM),
        ],
        out_shape=[
            jax.ShapeDtypeStruct((bn,), jnp.int32),
            jax.ShapeDtypeStruct((1, 1), jnp.float32),
        ],
        scratch_shapes=[
            pltpu.SMEM((1, 1), jnp.float32),
        ],
    )(z_e, embbf, e2)


_SC_CHUNK = 128


def _sc_gather(embedding, ind):
    bn = ind.shape[0]
    k, d = embedding.shape
    info = plsc.get_sparse_core_info()
    nw = info.num_cores * info.num_subcores
    b_per_w = bn // nw
    n_chunks = b_per_w // _SC_CHUNK
    mesh = plsc.VectorSubcoreMesh(core_axis_name="c", subcore_axis_name="s")

    @functools.partial(
        pl.kernel,
        mesh=mesh,
        out_type=jax.ShapeDtypeStruct((bn, d), jnp.float32),
        scratch_types=[
            pltpu.VMEM((_SC_CHUNK,), jnp.int32),
            pltpu.VMEM((_SC_CHUNK,), jnp.int32),
            pltpu.VMEM((_SC_CHUNK, d), jnp.float32),
            pltpu.VMEM((_SC_CHUNK, d), jnp.float32),
            pltpu.SemaphoreType.DMA,
            pltpu.SemaphoreType.DMA,
        ],
    )
    def gather_kernel(table_hbm, idx_hbm, out_hbm,
                      idx_a, idx_b, rows_a, rows_b, sem_a, sem_b):
        wid = lax.axis_index("s") * info.num_cores + lax.axis_index("c")
        base = wid * b_per_w
        idx_v = (idx_a, idx_b)
        rows_v = (rows_a, rows_b)
        sems = (sem_a, sem_b)
        pltpu.sync_copy(idx_hbm.at[pl.ds(base, _SC_CHUNK)], idx_a)
        pltpu.async_copy(table_hbm.at[idx_a], rows_a, sem_a)
        for c in range(n_chunks):
            cur = c % 2
            nxt = (c + 1) % 2
            if c + 1 < n_chunks:
                off_n = base + (c + 1) * _SC_CHUNK
                pltpu.sync_copy(idx_hbm.at[pl.ds(off_n, _SC_CHUNK)],
                                idx_v[nxt])
                pltpu.async_copy(table_hbm.at[idx_v[nxt]], rows_v[nxt],
                                 sems[nxt])
            pltpu.make_async_copy(table_hbm.at[idx_v[cur]], rows_v[cur],
                                  sems[cur]).wait()
            off = base + c * _SC_CHUNK
            pltpu.sync_copy(rows_v[cur], out_hbm.at[pl.ds(off, _SC_CHUNK)])

    return gather_kernel(embedding, ind)


def kernel(z, embedding):
    b, n, d = z.shape
    bn = b * n
    z_e = z.reshape(bn, d)
    embbf, e2 = _prep_call(embedding)
    ind, loss_sum = _argmin_call(z_e, embbf, e2)
    z_q = _sc_gather(embedding, ind)
    latent_loss = loss_sum[0, 0] * (12.5 / (bn * d))
    out = z_q.reshape(b, n * d)
    return (out, latent_loss)
